# baseline (speedup 1.0000x reference)
import functools

import jax
import jax.numpy as jnp
from jax import lax
from jax.experimental import pallas as pl
from jax.experimental.pallas import tpu as pltpu
from jax.experimental.pallas import tpu_sc as plsc

NN = 10000
NNP = 10240
E = 320000
D = 128
V = 100000
T = 128
NC, NS = 2, 16
NW = NC * NS
FPW = D // NW
BPW = NNP // NW
GCH = 64
NGC = BPW // GCH
EPW = E // NW
ECH = 8000
NEC = E // ECH
EPW2 = E // NS
HCAP2 = 1024
HCAP = 512
PCH = 512
CAPL = 8192
PAD = NNP - 1
TPAD = T
TST = T + 8

_mesh = plsc.VectorSubcoreMesh(
    core_axis_name="c", subcore_axis_name="s", num_cores=NC, num_subcores=NS
)
_sc_params = pltpu.CompilerParams(needs_layout_passes=False)


def _wid():
    return lax.axis_index("s") * NC + lax.axis_index("c")


def _prep_body(emb, y_r, src_e, dst_e, n_idx, zeros,
               rows_out, degp_out, l2s_out, l2d_out, head_out, cnt2_out,
               canon_out, l1s_out, l1d_out, cnt1_out,
               idx_v, rows_v, src_v, dst_v, deg_v, tpos, cs, cd,
               cnt_v, n_v, canon_v, shs, shc, sem):
    c = lax.axis_index("c")
    s = lax.axis_index("s")
    w = s * NC + c
    base = w * BPW
    pltpu.sync_copy(y_r.at[w], idx_v)
    for q in range(NGC):
        pltpu.async_copy(emb.at[idx_v.at[q]],
                         rows_v.at[pl.ds(q * GCH, GCH)], sem).wait()
    pltpu.sync_copy(rows_v, rows_out.at[pl.ds(base, BPW)])

    neg1 = jnp.full((16,), -1, jnp.int32)

    @plsc.parallel_loop(0, NNP // 16, unroll=8)
    def _tinit(g):
        tpos[pl.ds(g * 16, 16)] = neg1

    pltpu.sync_copy(n_idx, n_v)
    ones = jnp.ones((16,), jnp.float32)
    iota = lax.iota(jnp.int32, 16)
    for g in range(T // 16):
        t16 = n_v[pl.ds(g * 16, 16)]
        plsc.store_scatter(tpos, [t16], iota + (g * 16))
    for g in range(T // 16):
        t16 = n_v[pl.ds(g * 16, 16)]
        canon_v[pl.ds(g * 16, 16)] = plsc.load_gather(tpos, [t16])

    @pl.when(w == 0)
    def _():
        pltpu.sync_copy(canon_v, canon_out)

    padv = jnp.full((16,), PAD, jnp.int32)
    tpadv = jnp.full((16,), TPAD, jnp.int32)

    @plsc.parallel_loop(0, (EPW2 + 16) // 16, unroll=8)
    def _fill(g):
        cs[pl.ds(g * 16, 16)] = padv
        cd[pl.ds(g * 16, 16)] = tpadv

    pltpu.sync_copy(zeros.at[pl.ds(0, NNP)], deg_v)
    m = jnp.int32(0)
    for q in range(2):
        pltpu.sync_copy(src_e.at[pl.ds(s * EPW2 + q * EPW, EPW)], src_v)
        pltpu.sync_copy(dst_e.at[pl.ds(s * EPW2 + q * EPW, EPW)], dst_v)
        dval = jnp.broadcast_to((c == q).astype(jnp.float32), (16,))

        @plsc.parallel_loop(0, EPW // 16, unroll=4, carry=m)
        def _scan(g, mm):
            off = g * 16
            s16 = src_v[pl.ds(off, 16)]
            d16 = dst_v[pl.ds(off, 16)]
            plsc.addupdate_scatter(deg_v, [d16], dval)
            pv = plsc.load_gather(tpos, [d16])
            msk = pv >= 0
            plsc.store_compressed(cs.at[pl.ds(mm, 16)], s16, mask=msk)
            plsc.store_compressed(cd.at[pl.ds(mm, 16)], pv, mask=msk)
            return mm + jnp.sum(msk.astype(jnp.int32))

        m = _scan

    pltpu.sync_copy(deg_v, degp_out.at[pl.ds(w * NNP, NNP)])
    pltpu.sync_copy(cs.at[pl.ds(0, EPW2)], l2s_out.at[pl.ds(w * EPW2, EPW2)])
    pltpu.sync_copy(cd.at[pl.ds(0, EPW2)], l2d_out.at[pl.ds(w * EPW2, EPW2)])
    hb = w * 2 * HCAP2
    pltpu.sync_copy(cs.at[pl.ds(0, HCAP2)], head_out.at[pl.ds(hb, HCAP2)])
    pltpu.sync_copy(cd.at[pl.ds(0, HCAP2)],
                    head_out.at[pl.ds(hb + HCAP2, HCAP2)])
    cnt_v[...] = jnp.where(iota == 0, m, 0)
    pltpu.sync_copy(cnt_v, cnt2_out.at[pl.ds(w * 16, 16)])

    pltpu.sync_copy(cs.at[pl.ds(0, HCAP2)], shs.at[pl.ds(s * HCAP2, HCAP2)])
    pltpu.sync_copy(cnt_v, shc.at[pl.ds(s * 16, 16)])
    plsc.subcore_barrier()
    pltpu.sync_copy(zeros.at[pl.ds(0, NNP)], deg_v)
    pltpu.sync_copy(shc, cs.at[pl.ds(0, NS * 16)])
    for g in range(T // 16):
        t16 = n_v[pl.ds(g * 16, 16)]
        plsc.store_scatter(deg_v, [t16], ones)
    for v in range(NS):
        mv = jnp.sum(cs[pl.ds(v * 16, 16)])
        mmv = jnp.minimum(mv, HCAP2)
        pltpu.sync_copy(shs.at[pl.ds(v * HCAP2, HCAP2)],
                        src_v.at[pl.ds(0, HCAP2)])

        @plsc.parallel_loop(0, (mmv + 15) // 16, unroll=2)
        def _fm(g):
            s16 = src_v[pl.ds(g * 16, 16)]
            plsc.store_scatter(deg_v, [s16], ones)

    def ovf(v, cc):
        mv = jnp.sum(cs[pl.ds(v * 16, 16)])

        @pl.when(mv > HCAP2)
        def _():
            def part(p, c2):
                off0 = (v * NC + c) * EPW2 + HCAP2 + p * PCH
                pltpu.sync_copy(l2s_out.at[pl.ds(off0, PCH)],
                                dst_v.at[pl.ds(0, PCH)])
                rem = jnp.minimum(mv - HCAP2 - p * PCH, PCH)

                def grp2(g, c3):
                    s16 = dst_v[pl.ds(g * 16, 16)]
                    plsc.store_scatter(deg_v, [s16], ones)
                    return c3

                lax.fori_loop(0, (rem + 15) // 16, grp2, 0)
                return c2

            lax.fori_loop(0, (mv - HCAP2 + PCH - 1) // PCH, part, 0)

        return cc

    lax.fori_loop(0, NS, ovf, 0)

    @plsc.parallel_loop(0, (EPW + 16) // 16, unroll=8)
    def _fill1(g):
        cs[pl.ds(g * 16, 16)] = padv
        cd[pl.ds(g * 16, 16)] = padv

    pltpu.sync_copy(src_e.at[pl.ds(w * EPW, EPW)], src_v)
    pltpu.sync_copy(dst_e.at[pl.ds(w * EPW, EPW)], dst_v)

    @plsc.parallel_loop(0, EPW // 16, unroll=4, carry=jnp.int32(0))
    def _cmp(g, mm):
        off = g * 16
        s16 = src_v[pl.ds(off, 16)]
        d16 = dst_v[pl.ds(off, 16)]
        fv = plsc.load_gather(deg_v, [d16])
        msk = fv > 0.0
        plsc.store_compressed(cs.at[pl.ds(mm, 16)], s16, mask=msk)
        plsc.store_compressed(cd.at[pl.ds(mm, 16)], d16, mask=msk)
        return mm + jnp.sum(msk.astype(jnp.int32))

    m1 = _cmp
    pltpu.sync_copy(cs.at[pl.ds(0, EPW)], l1s_out.at[pl.ds(w * EPW, EPW)])
    pltpu.sync_copy(cd.at[pl.ds(0, EPW)], l1d_out.at[pl.ds(w * EPW, EPW)])
    cnt_v[...] = jnp.where(iota == 0, m1, 0)
    pltpu.sync_copy(cnt_v, cnt1_out.at[pl.ds(w * 16, 16)])


_prep = functools.partial(
    pl.kernel,
    out_type=[
        jax.ShapeDtypeStruct((NNP, D), jnp.float32),
        jax.ShapeDtypeStruct((NW * NNP,), jnp.float32),
        jax.ShapeDtypeStruct((NW * EPW2,), jnp.int32),
        jax.ShapeDtypeStruct((NW * EPW2,), jnp.int32),
        jax.ShapeDtypeStruct((NW * 2 * HCAP2,), jnp.int32),
        jax.ShapeDtypeStruct((NW * 16,), jnp.int32),
        jax.ShapeDtypeStruct((T,), jnp.int32),
        jax.ShapeDtypeStruct((NW * EPW,), jnp.int32),
        jax.ShapeDtypeStruct((NW * EPW,), jnp.int32),
        jax.ShapeDtypeStruct((NW * 16,), jnp.int32),
    ],
    mesh=_mesh,
    scratch_types=[
        pltpu.VMEM((NGC, GCH), jnp.int32),
        pltpu.VMEM((BPW, D), jnp.float32),
        pltpu.VMEM((EPW,), jnp.int32),
        pltpu.VMEM((EPW,), jnp.int32),
        pltpu.VMEM((NNP,), jnp.float32),
        pltpu.VMEM((NNP,), jnp.int32),
        pltpu.VMEM((EPW2 + 16,), jnp.int32),
        pltpu.VMEM((EPW2 + 16,), jnp.int32),
        pltpu.VMEM((16,), jnp.int32),
        pltpu.VMEM((T,), jnp.int32),
        pltpu.VMEM((T,), jnp.int32),
        pltpu.VMEM_SHARED((NS * HCAP2,), jnp.int32),
        pltpu.VMEM_SHARED((NS * 16,), jnp.int32),
        pltpu.SemaphoreType.DMA,
    ],
    compiler_params=_sc_params,
)(_prep_body)


def _spmm_body(xw_t, l1s, l1d, cnts1, l2s, l2d, cnts2, n_idx, canon,
               dinv, b1, zeros, out,
               xs, acc, dinv_v, b1_v, src_v0, src_v1, dst_v0, dst_v1,
               cv, cv2, acc2, n_v2, canon_v, sel_v, sems, semd):
    w = _wid()
    pltpu.sync_copy(xw_t.at[pl.ds(w * FPW * NNP, FPW * NNP)], xs)
    pltpu.sync_copy(dinv, dinv_v)
    pltpu.sync_copy(b1, b1_v)
    pltpu.sync_copy(zeros, acc)
    pltpu.sync_copy(cnts1, cv)

    bufs = ((src_v0, dst_v0), (src_v1, dst_v1))
    for b in range(2):
        pltpu.async_copy(l1s.at[pl.ds(b * EPW, CAPL)], bufs[b][0],
                         sems.at[b])
        pltpu.async_copy(l1d.at[pl.ds(b * EPW, CAPL)], bufs[b][1],
                         semd.at[b])

    @pl.loop(0, NW, step=2)
    def _chunks(k):
        for b in range(2):
            kk = k + b
            sv, dv_ = bufs[b]
            pltpu.make_async_copy(l1s.at[pl.ds(kk * EPW, CAPL)],
                                  sv, sems.at[b]).wait()
            pltpu.make_async_copy(l1d.at[pl.ds(kk * EPW, CAPL)],
                                  dv_, semd.at[b]).wait()
            m = jnp.sum(cv[pl.ds(kk * 16, 16)])
            mm = jnp.minimum(m, CAPL)

            @plsc.parallel_loop(0, (mm + 15) // 16, unroll=8)
            def _group(g):
                off = g * 16
                s16 = sv[pl.ds(off, 16)]
                d16 = dv_[pl.ds(off, 16)]
                nv = (plsc.load_gather(dinv_v, [s16])
                      * plsc.load_gather(dinv_v, [d16]))
                for j in range(FPW):
                    xv = plsc.load_gather(xs, [s16 + (j * NNP)])
                    plsc.addupdate_scatter(acc, [d16 + (j * NNP)], xv * nv)

            @pl.when(kk + 2 < NW)
            def _prefetch():
                pltpu.async_copy(l1s.at[pl.ds((kk + 2) * EPW, CAPL)],
                                 sv, sems.at[b])
                pltpu.async_copy(l1d.at[pl.ds((kk + 2) * EPW, CAPL)],
                                 dv_, semd.at[b])

    def ov(v, c):
        m = jnp.sum(cv[pl.ds(v * 16, 16)])

        @pl.when(m > CAPL)
        def _():
            def part(p, c2):
                off0 = v * EPW + CAPL + p * PCH
                pltpu.sync_copy(l1s.at[pl.ds(off0, PCH)],
                                src_v0.at[pl.ds(0, PCH)])
                pltpu.sync_copy(l1d.at[pl.ds(off0, PCH)],
                                dst_v0.at[pl.ds(0, PCH)])
                rem = jnp.minimum(m - CAPL - p * PCH, PCH)

                def grp2(g, c3):
                    off = g * 16
                    s16 = src_v0[pl.ds(off, 16)]
                    d16 = dst_v0[pl.ds(off, 16)]
                    nv = (plsc.load_gather(dinv_v, [s16])
                          * plsc.load_gather(dinv_v, [d16]))
                    for j in range(FPW):
                        xv = plsc.load_gather(xs, [s16 + (j * NNP)])
                        plsc.addupdate_scatter(acc, [d16 + (j * NNP)],
                                               xv * nv)
                    return c3

                lax.fori_loop(0, (rem + 15) // 16, grp2, 0)
                return c2

            lax.fori_loop(0, (m - CAPL + PCH - 1) // PCH, part, 0)

        return c

    lax.fori_loop(0, NW, ov, 0)

    @plsc.parallel_loop(0, NNP // 16, unroll=4)
    def _ep(g):
        off = g * 16
        dv = dinv_v[pl.ds(off, 16)]
        d2 = dv * dv
        for j in range(FPW):
            o = j * NNP + off
            bj = plsc.load_gather(b1_v, [jnp.full((16,), w * FPW + j,
                                                  jnp.int32)])
            v = acc[pl.ds(o, 16)] + d2 * xs[pl.ds(o, 16)] + bj
            acc[pl.ds(o, 16)] = jnp.maximum(v, 0.15 * v)

    pltpu.sync_copy(cnts2, cv2)
    pltpu.sync_copy(n_idx, n_v2.at[pl.ds(0, T)])
    pltpu.sync_copy(canon, canon_v)
    padv = jnp.full((16,), PAD, jnp.int32)
    n_v2[pl.ds(T, 16)] = padv

    @plsc.parallel_loop(0, (FPW * TST + 15) // 16, unroll=2)
    def _z2(g):
        acc2[pl.ds(g * 16, 16)] = jnp.zeros((16,), jnp.float32)

    def l2_group(sref, sbase, pref, pbase, g):
        s16 = sref[pl.ds(sbase + g * 16, 16)]
        p16 = pref[pl.ds(pbase + g * 16, 16)]
        dn16 = plsc.load_gather(n_v2, [p16])
        nv = (plsc.load_gather(dinv_v, [s16])
              * plsc.load_gather(dinv_v, [dn16]))
        for j in range(FPW):
            hv = plsc.load_gather(acc, [s16 + (j * NNP)])
            plsc.addupdate_scatter(acc2, [p16 + (j * TST)], hv * nv)

    for b in range(2):
        pltpu.async_copy(l2s.at[pl.ds((2 * b) * EPW2, HCAP2)],
                         bufs[b][0].at[pl.ds(0, HCAP2)], sems.at[b])
        pltpu.async_copy(l2d.at[pl.ds((2 * b) * EPW2, HCAP2)],
                         bufs[b][1].at[pl.ds(0, HCAP2)], semd.at[b])

    @pl.loop(0, NS, step=2)
    def _l2chunks(k):
        for b in range(2):
            kk = k + b
            sv, dv_ = bufs[b]
            pltpu.make_async_copy(l2s.at[pl.ds((2 * kk) * EPW2, HCAP2)],
                                  sv.at[pl.ds(0, HCAP2)], sems.at[b]).wait()
            pltpu.make_async_copy(l2d.at[pl.ds((2 * kk) * EPW2, HCAP2)],
                                  dv_.at[pl.ds(0, HCAP2)], semd.at[b]).wait()
            m = jnp.sum(cv2[pl.ds((2 * kk) * 16, 16)])
            mm = jnp.minimum(m, HCAP2)

            @plsc.parallel_loop(0, (mm + 15) // 16, unroll=4)
            def _g2(g):
                l2_group(sv, 0, dv_, 0, g)

            @pl.when(kk + 2 < NS)
            def _pf2():
                pltpu.async_copy(l2s.at[pl.ds((2 * (kk + 2)) * EPW2, HCAP2)],
                                 sv.at[pl.ds(0, HCAP2)], sems.at[b])
                pltpu.async_copy(l2d.at[pl.ds((2 * (kk + 2)) * EPW2, HCAP2)],
                                 dv_.at[pl.ds(0, HCAP2)], semd.at[b])

    def ov2(v, c):
        m = jnp.sum(cv2[pl.ds((2 * v) * 16, 16)])

        @pl.when(m > HCAP2)
        def _():
            def part(p, c2):
                off0 = (2 * v) * EPW2 + HCAP2 + p * PCH
                pltpu.sync_copy(l2s.at[pl.ds(off0, PCH)],
                                src_v0.at[pl.ds(0, PCH)])
                pltpu.sync_copy(l2d.at[pl.ds(off0, PCH)],
                                dst_v0.at[pl.ds(0, PCH)])
                rem = jnp.minimum(m - HCAP2 - p * PCH, PCH)

                def grp2(g, c3):
                    l2_group(src_v0, 0, dst_v0, 0, g)
                    return c3

                lax.fori_loop(0, (rem + 15) // 16, grp2, 0)
                return c2

            lax.fori_loop(0, (m - HCAP2 + PCH - 1) // PCH, part, 0)

        return c

    lax.fori_loop(0, NS, ov2, 0)

    for g in range(T // 16):
        t16 = n_v2[pl.ds(g * 16, 16)]
        c16 = canon_v[pl.ds(g * 16, 16)]
        dv = plsc.load_gather(dinv_v, [t16])
        d2 = dv * dv
        for j in range(FPW):
            av = plsc.load_gather(acc2, [c16 + (j * TST)])
            hv = plsc.load_gather(acc, [t16 + (j * NNP)])
            sel_v[pl.ds(j * T + g * 16, 16)] = av + d2 * hv
    pltpu.sync_copy(sel_v, out.at[pl.ds(w * FPW * T, FPW * T)])


_spmm_full = functools.partial(
    pl.kernel,
    out_type=jax.ShapeDtypeStruct((NW * FPW * T,), jnp.float32),
    mesh=_mesh,
    scratch_types=[
        pltpu.VMEM((FPW * NNP,), jnp.float32),
        pltpu.VMEM((FPW * NNP,), jnp.float32),
        pltpu.VMEM((NNP,), jnp.float32),
        pltpu.VMEM((D,), jnp.float32),
        pltpu.VMEM((CAPL,), jnp.int32),
        pltpu.VMEM((CAPL,), jnp.int32),
        pltpu.VMEM((CAPL,), jnp.int32),
        pltpu.VMEM((CAPL,), jnp.int32),
        pltpu.VMEM((NW * 16,), jnp.int32),
        pltpu.VMEM((NW * 16,), jnp.int32),
        pltpu.VMEM((FPW * TST,), jnp.float32),
        pltpu.VMEM((T + 16,), jnp.int32),
        pltpu.VMEM((T,), jnp.int32),
        pltpu.VMEM((FPW * T,), jnp.float32),
        pltpu.SemaphoreType.DMA((2,)),
        pltpu.SemaphoreType.DMA((2,)),
    ],
    compiler_params=_sc_params,
)(_spmm_body)


def _ln_w1_body(rows_ref, w1_ref, degp_ref, out_ref, dinv_ref):
    r = rows_ref[:]
    mu = jnp.mean(r, axis=-1, keepdims=True)
    var = jnp.mean((r - mu) ** 2, axis=-1, keepdims=True)
    x = (r - mu) * lax.rsqrt(var + 1e-5)
    out_ref[:] = lax.dot_general(
        w1_ref[:], x, (((1,), (1,)), ((), ())),
        preferred_element_type=jnp.float32)

    @pl.when(pl.program_id(0) == 0)
    def _():
        deg = jnp.sum(degp_ref[:], axis=0) + 1.0
        dinv_ref[:] = lax.rsqrt(deg)


def _head_body(sel_ref, w2_ref, b2_ref, wout_ref, bout_ref, out_ref,
               tmp_ref):
    @pl.when(pl.program_id(0) == 0)
    def _():
        tmp_ref[:] = lax.dot_general(
            sel_ref[:], w2_ref[:], (((0,), (1,)), ((), ())),
            preferred_element_type=jnp.float32) + b2_ref[:]

    out_ref[:] = lax.dot_general(
        tmp_ref[:], wout_ref[:], (((1,), (1,)), ((), ())),
        preferred_element_type=jnp.float32) + bout_ref[:]


_NB = 1024
_VB = 2048


def kernel(edge_index, N, y, emb, W1, b1, W2, b2, Wout, bout):
    src = edge_index[0].astype(jnp.int32)
    dst = edge_index[1].astype(jnp.int32)
    y_pad = jnp.concatenate(
        [y.astype(jnp.int32), jnp.zeros((NNP - NN,), jnp.int32)]
    ).reshape(NW, NGC, GCH)
    n_idx = N.astype(jnp.int32)
    zeros = jnp.zeros((FPW * NNP,), jnp.float32)

    (rows, deg_p, l2s, l2d, head, cnts, canon,
     l1s, l1d, cnts1) = _prep(emb, y_pad, src, dst, n_idx, zeros)

    xw1_t, dinv = pl.pallas_call(
        _ln_w1_body,
        grid=(NNP // _NB,),
        in_specs=[
            pl.BlockSpec((_NB, D), lambda i: (i, 0)),
            pl.BlockSpec((D, D), lambda i: (0, 0)),
            pl.BlockSpec((NW, NNP // D, D), lambda i: (0, 0, 0)),
        ],
        out_specs=[
            pl.BlockSpec((D, _NB), lambda i: (0, i)),
            pl.BlockSpec((NNP // D, D), lambda i: (0, 0)),
        ],
        out_shape=[
            jax.ShapeDtypeStruct((D, NNP), jnp.float32),
            jax.ShapeDtypeStruct((NNP // D, D), jnp.float32),
        ],
    )(rows, W1, deg_p.reshape(NW, NNP // D, D))
    dinv = dinv.reshape(NNP)

    sel = _spmm_full(xw1_t.reshape(D * NNP), l1s, l1d, cnts1,
                     l2s, l2d, cnts, n_idx, canon, dinv, b1, zeros)

    out = pl.pallas_call(
        _head_body,
        grid=(pl.cdiv(V, _VB),),
        in_specs=[
            pl.BlockSpec((D, T), lambda i: (0, 0)),
            pl.BlockSpec((D, D), lambda i: (0, 0)),
            pl.BlockSpec((1, D), lambda i: (0, 0)),
            pl.BlockSpec((_VB, D), lambda i: (i, 0)),
            pl.BlockSpec((1, _VB), lambda i: (0, i)),
        ],
        out_specs=pl.BlockSpec((T, _VB), lambda i: (0, i)),
        out_shape=jax.ShapeDtypeStruct((T, V), jnp.float32),
        scratch_shapes=[pltpu.VMEM((T, D), jnp.float32)],
    )(sel.reshape(NW * FPW, T).reshape(D, T), W2, b2.reshape(1, D),
      Wout, bout.reshape(1, V))

    return out


# speedup vs baseline: 1.0214x; 1.0214x over previous
import functools

import jax
import jax.numpy as jnp
from jax import lax
from jax.experimental import pallas as pl
from jax.experimental.pallas import tpu as pltpu
from jax.experimental.pallas import tpu_sc as plsc

NN = 10000
NNP = 10240
E = 320000
D = 128
V = 100000
T = 128
NC, NS = 2, 16
NW = NC * NS
FPW = D // NW
BPW = NNP // NW
GCH = 64
NGC = BPW // GCH
EPW = E // NW
ECH = 8000
NEC = E // ECH
HCAP = 512
PCH = 512
CAPL = 8192
PAD = NNP - 1
TPAD = T
TST = T + 8

_mesh = plsc.VectorSubcoreMesh(
    core_axis_name="c", subcore_axis_name="s", num_cores=NC, num_subcores=NS
)
_sc_params = pltpu.CompilerParams(needs_layout_passes=False)


def _wid():
    return lax.axis_index("s") * NC + lax.axis_index("c")


def _prep_body(emb, y_r, src_e, dst_e, n_idx, zeros,
               rows_out, degp_out, l2s_out, l2d_out, head_out, cnt_out,
               canon_out,
               idx_v, rows_v, src_v, dst_v, deg_v, tmask, cs, cd,
               cnt_v, n_v, canon_v, sem):
    w = _wid()
    base = w * BPW
    pltpu.sync_copy(y_r.at[w], idx_v)
    descs = [pltpu.async_copy(emb.at[idx_v.at[q]],
                              rows_v.at[pl.ds(q * GCH, GCH)], sem)
             for q in range(NGC)]
    for d in descs:
        d.wait()
    pltpu.sync_copy(rows_v, rows_out.at[pl.ds(base, BPW)])

    neg1 = jnp.full((16,), -1, jnp.int32)

    @plsc.parallel_loop(0, NNP // 16, unroll=8)
    def _tinit(g):
        tmask[pl.ds(g * 16, 16)] = neg1

    pltpu.sync_copy(n_idx, n_v)
    ones = jnp.ones((16,), jnp.float32)
    iota = lax.iota(jnp.int32, 16)
    for g in range(T // 16):
        t16 = n_v[pl.ds(g * 16, 16)]
        plsc.store_scatter(tmask, [t16], iota + (g * 16))
    for g in range(T // 16):
        t16 = n_v[pl.ds(g * 16, 16)]
        canon_v[pl.ds(g * 16, 16)] = plsc.load_gather(tmask, [t16])

    padv = jnp.full((16,), PAD, jnp.int32)
    tpadv = jnp.full((16,), TPAD, jnp.int32)

    @plsc.parallel_loop(0, (EPW + 16) // 16, unroll=8)
    def _fill(g):
        cs[pl.ds(g * 16, 16)] = padv
        cd[pl.ds(g * 16, 16)] = tpadv

    pltpu.sync_copy(zeros.at[pl.ds(0, NNP)], deg_v)
    pltpu.sync_copy(src_e.at[pl.ds(w * EPW, EPW)], src_v)
    pltpu.sync_copy(dst_e.at[pl.ds(w * EPW, EPW)], dst_v)

    @plsc.parallel_loop(0, EPW // 16, unroll=4, carry=jnp.int32(0))
    def _deg(g, m):
        off = g * 16
        s16 = src_v[pl.ds(off, 16)]
        d16 = dst_v[pl.ds(off, 16)]
        plsc.addupdate_scatter(deg_v, [d16], ones)
        pv = plsc.load_gather(tmask, [d16])
        msk = pv >= 0
        plsc.store_compressed(cs.at[pl.ds(m, 16)], s16, mask=msk)
        plsc.store_compressed(cd.at[pl.ds(m, 16)], pv, mask=msk)
        return m + jnp.sum(msk.astype(jnp.int32))

    m = _deg
    pltpu.sync_copy(deg_v, degp_out.at[pl.ds(w * NNP, NNP)])
    pltpu.sync_copy(cs.at[pl.ds(0, EPW)], l2s_out.at[pl.ds(w * EPW, EPW)])
    pltpu.sync_copy(cd.at[pl.ds(0, EPW)], l2d_out.at[pl.ds(w * EPW, EPW)])
    hb = w * 2 * HCAP
    pltpu.sync_copy(cs.at[pl.ds(0, HCAP)], head_out.at[pl.ds(hb, HCAP)])
    pltpu.sync_copy(cd.at[pl.ds(0, HCAP)], head_out.at[pl.ds(hb + HCAP, HCAP)])
    @pl.when(w == 0)
    def _():
        pltpu.sync_copy(canon_v, canon_out)

    cnt_v[...] = jnp.where(iota == 0, m, 0)
    pltpu.sync_copy(cnt_v, cnt_out.at[pl.ds(w * 16, 16)])


_prep = functools.partial(
    pl.kernel,
    out_type=[
        jax.ShapeDtypeStruct((NNP, D), jnp.float32),
        jax.ShapeDtypeStruct((NW * NNP,), jnp.float32),
        jax.ShapeDtypeStruct((NW * EPW,), jnp.int32),
        jax.ShapeDtypeStruct((NW * EPW,), jnp.int32),
        jax.ShapeDtypeStruct((NW * 2 * HCAP,), jnp.int32),
        jax.ShapeDtypeStruct((NW * 16,), jnp.int32),
        jax.ShapeDtypeStruct((T,), jnp.int32),
    ],
    mesh=_mesh,
    scratch_types=[
        pltpu.VMEM((NGC, GCH), jnp.int32),
        pltpu.VMEM((BPW, D), jnp.float32),
        pltpu.VMEM((EPW,), jnp.int32),
        pltpu.VMEM((EPW,), jnp.int32),
        pltpu.VMEM((NNP,), jnp.float32),
        pltpu.VMEM((NNP,), jnp.int32),
        pltpu.VMEM((EPW + 16,), jnp.int32),
        pltpu.VMEM((EPW + 16,), jnp.int32),
        pltpu.VMEM((16,), jnp.int32),
        pltpu.VMEM((T,), jnp.int32),
        pltpu.VMEM((T,), jnp.int32),
        pltpu.SemaphoreType.DMA,
    ],
    compiler_params=_sc_params,
)(_prep_body)


def _frontier_body(l2s, head, cnts, n_idx, src_e, dst_e, zeros,
                   l1s_out, l1d_out, cnt_out,
                   fmask, head_v, cv, n_v, src_v, dst_v, cs, cd,
                   cnt_v, ovbuf):
    w = _wid()
    pltpu.sync_copy(zeros.at[pl.ds(0, NNP)], fmask)
    pltpu.sync_copy(n_idx, n_v)
    pltpu.sync_copy(cnts, cv)
    pltpu.sync_copy(head, head_v)
    ones = jnp.ones((16,), jnp.float32)
    for g in range(T // 16):
        t16 = n_v[pl.ds(g * 16, 16)]
        plsc.store_scatter(fmask, [t16], ones)
    for v in range(NW):
        m = jnp.sum(cv[pl.ds(v * 16, 16)])
        mm = jnp.minimum(m, HCAP)

        @plsc.parallel_loop(0, (mm + 15) // 16, unroll=2)
        def _sc(g, v=v):
            s16 = head_v[pl.ds(v * 2 * HCAP + g * 16, 16)]
            plsc.store_scatter(fmask, [s16], ones)

    def ov(v, c):
        m = jnp.sum(cv[pl.ds(v * 16, 16)])

        @pl.when(m > HCAP)
        def _():
            def part(p, c2):
                pltpu.sync_copy(l2s.at[pl.ds(v * EPW + p * PCH, PCH)], ovbuf)
                rem = jnp.minimum(m - p * PCH, PCH)

                def grp2(g, c3):
                    s16 = ovbuf[pl.ds(g * 16, 16)]
                    plsc.store_scatter(fmask, [s16], ones)
                    return c3

                lax.fori_loop(0, (rem + 15) // 16, grp2, 0)
                return c2

            lax.fori_loop(1, (m + PCH - 1) // PCH, part, 0)

        return c

    lax.fori_loop(0, NW, ov, 0)

    padv = jnp.full((16,), PAD, jnp.int32)

    @plsc.parallel_loop(0, (EPW + 16) // 16, unroll=8)
    def _fill(g):
        cs[pl.ds(g * 16, 16)] = padv
        cd[pl.ds(g * 16, 16)] = padv

    pltpu.sync_copy(src_e.at[pl.ds(w * EPW, EPW)], src_v)
    pltpu.sync_copy(dst_e.at[pl.ds(w * EPW, EPW)], dst_v)

    @plsc.parallel_loop(0, EPW // 16, unroll=4, carry=jnp.int32(0))
    def _cmp(g, m):
        off = g * 16
        s16 = src_v[pl.ds(off, 16)]
        d16 = dst_v[pl.ds(off, 16)]
        fv = plsc.load_gather(fmask, [d16])
        msk = fv > 0.0
        plsc.store_compressed(cs.at[pl.ds(m, 16)], s16, mask=msk)
        plsc.store_compressed(cd.at[pl.ds(m, 16)], d16, mask=msk)
        return m + jnp.sum(msk.astype(jnp.int32))

    m = _cmp
    pltpu.sync_copy(cs.at[pl.ds(0, EPW)], l1s_out.at[pl.ds(w * EPW, EPW)])
    pltpu.sync_copy(cd.at[pl.ds(0, EPW)], l1d_out.at[pl.ds(w * EPW, EPW)])
    iota = lax.iota(jnp.int32, 16)
    cnt_v[...] = jnp.where(iota == 0, m, 0)
    pltpu.sync_copy(cnt_v, cnt_out.at[pl.ds(w * 16, 16)])


_frontier = functools.partial(
    pl.kernel,
    out_type=[
        jax.ShapeDtypeStruct((NW * EPW,), jnp.int32),
        jax.ShapeDtypeStruct((NW * EPW,), jnp.int32),
        jax.ShapeDtypeStruct((NW * 16,), jnp.int32),
    ],
    mesh=_mesh,
    scratch_types=[
        pltpu.VMEM((NNP,), jnp.float32),
        pltpu.VMEM((NW * 2 * HCAP,), jnp.int32),
        pltpu.VMEM((NW * 16,), jnp.int32),
        pltpu.VMEM((T,), jnp.int32),
        pltpu.VMEM((EPW,), jnp.int32),
        pltpu.VMEM((EPW,), jnp.int32),
        pltpu.VMEM((EPW + 16,), jnp.int32),
        pltpu.VMEM((EPW + 16,), jnp.int32),
        pltpu.VMEM((16,), jnp.int32),
        pltpu.VMEM((PCH,), jnp.int32),
    ],
    compiler_params=_sc_params,
)(_frontier_body)


def _spmm_body(xw_t, l1s, l1d, cnts1, l2s, l2d, cnts2, n_idx, canon,
               dinv, b1, zeros, out,
               xs, acc, dinv_v, b1_v, src_v0, src_v1, dst_v0, dst_v1,
               cv, cv2, acc2, n_v2, canon_v, sel_v, sems, semd, psem):
    w = _wid()
    pdescs = [
        pltpu.async_copy(xw_t.at[pl.ds(w * FPW * NNP, FPW * NNP)], xs,
                         psem.at[0]),
        pltpu.async_copy(dinv, dinv_v, psem.at[1]),
        pltpu.async_copy(b1, b1_v, psem.at[2]),
        pltpu.async_copy(zeros, acc, psem.at[3]),
    ]
    pltpu.sync_copy(cnts1, cv)
    for d in pdescs:
        d.wait()

    bufs = ((src_v0, dst_v0), (src_v1, dst_v1))
    for b in range(2):
        pltpu.async_copy(l1s.at[pl.ds(b * EPW, CAPL)], bufs[b][0],
                         sems.at[b])
        pltpu.async_copy(l1d.at[pl.ds(b * EPW, CAPL)], bufs[b][1],
                         semd.at[b])

    @pl.loop(0, NW, step=2)
    def _chunks(k):
        for b in range(2):
            kk = k + b
            sv, dv_ = bufs[b]
            pltpu.make_async_copy(l1s.at[pl.ds(kk * EPW, CAPL)],
                                  sv, sems.at[b]).wait()
            pltpu.make_async_copy(l1d.at[pl.ds(kk * EPW, CAPL)],
                                  dv_, semd.at[b]).wait()
            m = jnp.sum(cv[pl.ds(kk * 16, 16)])
            mm = jnp.minimum(m, CAPL)

            @plsc.parallel_loop(0, (mm + 15) // 16, unroll=8)
            def _group(g):
                off = g * 16
                s16 = sv[pl.ds(off, 16)]
                d16 = dv_[pl.ds(off, 16)]
                nv = (plsc.load_gather(dinv_v, [s16])
                      * plsc.load_gather(dinv_v, [d16]))
                for j in range(FPW):
                    xv = plsc.load_gather(xs, [s16 + (j * NNP)])
                    plsc.addupdate_scatter(acc, [d16 + (j * NNP)], xv * nv)

            @pl.when(kk + 2 < NW)
            def _prefetch():
                pltpu.async_copy(l1s.at[pl.ds((kk + 2) * EPW, CAPL)],
                                 sv, sems.at[b])
                pltpu.async_copy(l1d.at[pl.ds((kk + 2) * EPW, CAPL)],
                                 dv_, semd.at[b])

    def ov(v, c):
        m = jnp.sum(cv[pl.ds(v * 16, 16)])

        @pl.when(m > CAPL)
        def _():
            def part(p, c2):
                off0 = v * EPW + CAPL + p * PCH
                pltpu.sync_copy(l1s.at[pl.ds(off0, PCH)],
                                src_v0.at[pl.ds(0, PCH)])
                pltpu.sync_copy(l1d.at[pl.ds(off0, PCH)],
                                dst_v0.at[pl.ds(0, PCH)])
                rem = jnp.minimum(m - CAPL - p * PCH, PCH)

                def grp2(g, c3):
                    off = g * 16
                    s16 = src_v0[pl.ds(off, 16)]
                    d16 = dst_v0[pl.ds(off, 16)]
                    nv = (plsc.load_gather(dinv_v, [s16])
                          * plsc.load_gather(dinv_v, [d16]))
                    for j in range(FPW):
                        xv = plsc.load_gather(xs, [s16 + (j * NNP)])
                        plsc.addupdate_scatter(acc, [d16 + (j * NNP)],
                                               xv * nv)
                    return c3

                lax.fori_loop(0, (rem + 15) // 16, grp2, 0)
                return c2

            lax.fori_loop(0, (m - CAPL + PCH - 1) // PCH, part, 0)

        return c

    lax.fori_loop(0, NW, ov, 0)

    @plsc.parallel_loop(0, NNP // 16, unroll=4)
    def _ep(g):
        off = g * 16
        dv = dinv_v[pl.ds(off, 16)]
        d2 = dv * dv
        for j in range(FPW):
            o = j * NNP + off
            bj = plsc.load_gather(b1_v, [jnp.full((16,), w * FPW + j,
                                                  jnp.int32)])
            v = acc[pl.ds(o, 16)] + d2 * xs[pl.ds(o, 16)] + bj
            acc[pl.ds(o, 16)] = jnp.maximum(v, 0.15 * v)

    pltpu.sync_copy(cnts2, cv2)
    pltpu.sync_copy(n_idx, n_v2.at[pl.ds(0, T)])
    pltpu.sync_copy(canon, canon_v)
    padv = jnp.full((16,), PAD, jnp.int32)
    n_v2[pl.ds(T, 16)] = padv

    @plsc.parallel_loop(0, (FPW * TST + 15) // 16, unroll=2)
    def _z2(g):
        acc2[pl.ds(g * 16, 16)] = jnp.zeros((16,), jnp.float32)

    def l2_group(sref, sbase, pref, pbase, g):
        s16 = sref[pl.ds(sbase + g * 16, 16)]
        p16 = pref[pl.ds(pbase + g * 16, 16)]
        dn16 = plsc.load_gather(n_v2, [p16])
        nv = (plsc.load_gather(dinv_v, [s16])
              * plsc.load_gather(dinv_v, [dn16]))
        for j in range(FPW):
            hv = plsc.load_gather(acc, [s16 + (j * NNP)])
            plsc.addupdate_scatter(acc2, [p16 + (j * TST)], hv * nv)

    for b in range(2):
        pltpu.async_copy(l2s.at[pl.ds(b * EPW, HCAP)],
                         bufs[b][0].at[pl.ds(0, HCAP)], sems.at[b])
        pltpu.async_copy(l2d.at[pl.ds(b * EPW, HCAP)],
                         bufs[b][1].at[pl.ds(0, HCAP)], semd.at[b])

    @pl.loop(0, NW, step=2)
    def _l2chunks(k):
        for b in range(2):
            kk = k + b
            sv, dv_ = bufs[b]
            pltpu.make_async_copy(l2s.at[pl.ds(kk * EPW, HCAP)],
                                  sv.at[pl.ds(0, HCAP)], sems.at[b]).wait()
            pltpu.make_async_copy(l2d.at[pl.ds(kk * EPW, HCAP)],
                                  dv_.at[pl.ds(0, HCAP)], semd.at[b]).wait()
            m = jnp.sum(cv2[pl.ds(kk * 16, 16)])
            mm = jnp.minimum(m, HCAP)

            @plsc.parallel_loop(0, (mm + 15) // 16, unroll=4)
            def _g2(g):
                l2_group(sv, 0, dv_, 0, g)

            @pl.when(kk + 2 < NW)
            def _pf2():
                pltpu.async_copy(l2s.at[pl.ds((kk + 2) * EPW, HCAP)],
                                 sv.at[pl.ds(0, HCAP)], sems.at[b])
                pltpu.async_copy(l2d.at[pl.ds((kk + 2) * EPW, HCAP)],
                                 dv_.at[pl.ds(0, HCAP)], semd.at[b])

    def ov2(v, c):
        m = jnp.sum(cv2[pl.ds(v * 16, 16)])

        @pl.when(m > HCAP)
        def _():
            def part(p, c2):
                off0 = v * EPW + HCAP + p * PCH
                pltpu.sync_copy(l2s.at[pl.ds(off0, PCH)],
                                src_v0.at[pl.ds(0, PCH)])
                pltpu.sync_copy(l2d.at[pl.ds(off0, PCH)],
                                dst_v0.at[pl.ds(0, PCH)])
                rem = jnp.minimum(m - HCAP - p * PCH, PCH)

                def grp2(g, c3):
                    l2_group(src_v0, 0, dst_v0, 0, g)
                    return c3

                lax.fori_loop(0, (rem + 15) // 16, grp2, 0)
                return c2

            lax.fori_loop(0, (m - HCAP + PCH - 1) // PCH, part, 0)

        return c

    lax.fori_loop(0, NW, ov2, 0)

    for g in range(T // 16):
        t16 = n_v2[pl.ds(g * 16, 16)]
        c16 = canon_v[pl.ds(g * 16, 16)]
        dv = plsc.load_gather(dinv_v, [t16])
        d2 = dv * dv
        for j in range(FPW):
            av = plsc.load_gather(acc2, [c16 + (j * TST)])
            hv = plsc.load_gather(acc, [t16 + (j * NNP)])
            sel_v[pl.ds(j * T + g * 16, 16)] = av + d2 * hv
    pltpu.sync_copy(sel_v, out.at[pl.ds(w * FPW * T, FPW * T)])


_spmm_full = functools.partial(
    pl.kernel,
    out_type=jax.ShapeDtypeStruct((NW * FPW * T,), jnp.float32),
    mesh=_mesh,
    scratch_types=[
        pltpu.VMEM((FPW * NNP,), jnp.float32),
        pltpu.VMEM((FPW * NNP,), jnp.float32),
        pltpu.VMEM((NNP,), jnp.float32),
        pltpu.VMEM((D,), jnp.float32),
        pltpu.VMEM((CAPL,), jnp.int32),
        pltpu.VMEM((CAPL,), jnp.int32),
        pltpu.VMEM((CAPL,), jnp.int32),
        pltpu.VMEM((CAPL,), jnp.int32),
        pltpu.VMEM((NW * 16,), jnp.int32),
        pltpu.VMEM((NW * 16,), jnp.int32),
        pltpu.VMEM((FPW * TST,), jnp.float32),
        pltpu.VMEM((T + 16,), jnp.int32),
        pltpu.VMEM((T,), jnp.int32),
        pltpu.VMEM((FPW * T,), jnp.float32),
        pltpu.SemaphoreType.DMA((2,)),
        pltpu.SemaphoreType.DMA((2,)),
        pltpu.SemaphoreType.DMA((4,)),
    ],
    compiler_params=_sc_params,
)(_spmm_body)


def _ln_w1_body(rows_ref, w1_ref, degp_ref, out_ref, dinv_ref):
    r = rows_ref[:]
    mu = jnp.mean(r, axis=-1, keepdims=True)
    var = jnp.mean((r - mu) ** 2, axis=-1, keepdims=True)
    x = (r - mu) * lax.rsqrt(var + 1e-5)
    out_ref[:] = lax.dot_general(
        w1_ref[:], x, (((1,), (1,)), ((), ())),
        preferred_element_type=jnp.float32)

    @pl.when(pl.program_id(0) == 0)
    def _():
        deg = jnp.sum(degp_ref[:], axis=0) + 1.0
        dinv_ref[:] = lax.rsqrt(deg)


def _head_body(sel_ref, w2_ref, b2_ref, wout_ref, bout_ref, out_ref,
               tmp_ref):
    @pl.when(pl.program_id(0) == 0)
    def _():
        tmp_ref[:] = lax.dot_general(
            sel_ref[:], w2_ref[:], (((0,), (1,)), ((), ())),
            preferred_element_type=jnp.float32) + b2_ref[:]

    out_ref[:] = lax.dot_general(
        tmp_ref[:], wout_ref[:], (((1,), (1,)), ((), ())),
        preferred_element_type=jnp.float32) + bout_ref[:]


_NB = 1024
_VB = 2048


def kernel(edge_index, N, y, emb, W1, b1, W2, b2, Wout, bout):
    src = edge_index[0].astype(jnp.int32)
    dst = edge_index[1].astype(jnp.int32)
    y_pad = jnp.concatenate(
        [y.astype(jnp.int32), jnp.zeros((NNP - NN,), jnp.int32)]
    ).reshape(NW, NGC, GCH)
    n_idx = N.astype(jnp.int32)
    zeros = jnp.zeros((FPW * NNP,), jnp.float32)

    rows, deg_p, l2s, l2d, head, cnts, canon = _prep(emb, y_pad, src, dst,
                                                     n_idx, zeros)
    l1s, l1d, cnts1 = _frontier(l2s, head, cnts, n_idx, src, dst, zeros)

    xw1_t, dinv = pl.pallas_call(
        _ln_w1_body,
        grid=(NNP // _NB,),
        in_specs=[
            pl.BlockSpec((_NB, D), lambda i: (i, 0)),
            pl.BlockSpec((D, D), lambda i: (0, 0)),
            pl.BlockSpec((NW, NNP // D, D), lambda i: (0, 0, 0)),
        ],
        out_specs=[
            pl.BlockSpec((D, _NB), lambda i: (0, i)),
            pl.BlockSpec((NNP // D, D), lambda i: (0, 0)),
        ],
        out_shape=[
            jax.ShapeDtypeStruct((D, NNP), jnp.float32),
            jax.ShapeDtypeStruct((NNP // D, D), jnp.float32),
        ],
    )(rows, W1, deg_p.reshape(NW, NNP // D, D))
    dinv = dinv.reshape(NNP)

    sel = _spmm_full(xw1_t.reshape(D * NNP), l1s, l1d, cnts1,
                     l2s, l2d, cnts, n_idx, canon, dinv, b1, zeros)

    out = pl.pallas_call(
        _head_body,
        grid=(pl.cdiv(V, _VB),),
        in_specs=[
            pl.BlockSpec((D, T), lambda i: (0, 0)),
            pl.BlockSpec((D, D), lambda i: (0, 0)),
            pl.BlockSpec((1, D), lambda i: (0, 0)),
            pl.BlockSpec((_VB, D), lambda i: (i, 0)),
            pl.BlockSpec((1, _VB), lambda i: (0, i)),
        ],
        out_specs=pl.BlockSpec((T, _VB), lambda i: (0, i)),
        out_shape=jax.ShapeDtypeStruct((T, V), jnp.float32),
        scratch_shapes=[pltpu.VMEM((T, D), jnp.float32)],
    )(sel.reshape(NW * FPW, T).reshape(D, T), W2, b2.reshape(1, D),
      Wout, bout.reshape(1, V))

    return out


# speedup vs baseline: 1.0864x; 1.0636x over previous
import functools

import jax
import jax.numpy as jnp
from jax import lax
from jax.experimental import pallas as pl
from jax.experimental.pallas import tpu as pltpu
from jax.experimental.pallas import tpu_sc as plsc

NN = 10000
NNP = 10240
E = 320000
D = 128
V = 100000
T = 128
NC, NS = 2, 16
NW = NC * NS
FPW = D // NW
BPW = NNP // NW
GCH = 64
NGC = BPW // GCH
EPW = E // NW
ECH = 8000
NEC = E // ECH
HCAP = 512
PCH = 512
CAPL = 8192
PAD = NNP - 1
TPAD = T
TST = T + 8

_mesh = plsc.VectorSubcoreMesh(
    core_axis_name="c", subcore_axis_name="s", num_cores=NC, num_subcores=NS
)
_sc_params = pltpu.CompilerParams(needs_layout_passes=False)


def _wid():
    return lax.axis_index("s") * NC + lax.axis_index("c")


def _prep_body(emb, y_r, src_e, dst_e, n_idx, zeros,
               rows_out, degp_out, l2s_out, l2d_out, head_out, cnt_out,
               canon_out,
               idx_v, rows_v, src_v, dst_v, deg_v, tmask, cs, cd,
               cnt_v, n_v, canon_v, sem, esem):
    w = _wid()
    base = w * BPW
    pltpu.sync_copy(y_r.at[w], idx_v)
    edescs = [
        pltpu.async_copy(src_e.at[pl.ds(w * EPW, EPW)], src_v, esem.at[0]),
        pltpu.async_copy(dst_e.at[pl.ds(w * EPW, EPW)], dst_v, esem.at[1]),
        pltpu.async_copy(zeros.at[pl.ds(0, NNP)], deg_v, esem.at[2]),
    ]
    descs = [pltpu.async_copy(emb.at[idx_v.at[q]],
                              rows_v.at[pl.ds(q * GCH, GCH)], sem)
             for q in range(NGC)]
    for d in descs:
        d.wait()
    pltpu.sync_copy(rows_v, rows_out.at[pl.ds(base, BPW)])

    neg1 = jnp.full((16,), -1, jnp.int32)

    @plsc.parallel_loop(0, NNP // 16, unroll=8)
    def _tinit(g):
        tmask[pl.ds(g * 16, 16)] = neg1

    pltpu.sync_copy(n_idx, n_v)
    ones = jnp.ones((16,), jnp.float32)
    iota = lax.iota(jnp.int32, 16)
    for g in range(T // 16):
        t16 = n_v[pl.ds(g * 16, 16)]
        plsc.store_scatter(tmask, [t16], iota + (g * 16))
    for g in range(T // 16):
        t16 = n_v[pl.ds(g * 16, 16)]
        canon_v[pl.ds(g * 16, 16)] = plsc.load_gather(tmask, [t16])

    padv = jnp.full((16,), PAD, jnp.int32)
    tpadv = jnp.full((16,), TPAD, jnp.int32)

    @plsc.parallel_loop(0, (EPW + 16) // 16, unroll=8)
    def _fill(g):
        cs[pl.ds(g * 16, 16)] = padv
        cd[pl.ds(g * 16, 16)] = tpadv

    for d in edescs:
        d.wait()

    @plsc.parallel_loop(0, EPW // 16, unroll=4, carry=jnp.int32(0))
    def _deg(g, m):
        off = g * 16
        s16 = src_v[pl.ds(off, 16)]
        d16 = dst_v[pl.ds(off, 16)]
        plsc.addupdate_scatter(deg_v, [d16], ones)
        pv = plsc.load_gather(tmask, [d16])
        msk = pv >= 0
        plsc.store_compressed(cs.at[pl.ds(m, 16)], s16, mask=msk)
        plsc.store_compressed(cd.at[pl.ds(m, 16)], pv, mask=msk)
        return m + jnp.sum(msk.astype(jnp.int32))

    m = _deg
    pltpu.sync_copy(deg_v, degp_out.at[pl.ds(w * NNP, NNP)])
    pltpu.sync_copy(cs.at[pl.ds(0, EPW)], l2s_out.at[pl.ds(w * EPW, EPW)])
    pltpu.sync_copy(cd.at[pl.ds(0, EPW)], l2d_out.at[pl.ds(w * EPW, EPW)])
    hb = w * 2 * HCAP
    pltpu.sync_copy(cs.at[pl.ds(0, HCAP)], head_out.at[pl.ds(hb, HCAP)])
    pltpu.sync_copy(cd.at[pl.ds(0, HCAP)], head_out.at[pl.ds(hb + HCAP, HCAP)])
    @pl.when(w == 0)
    def _():
        pltpu.sync_copy(canon_v, canon_out)

    cnt_v[...] = jnp.where(iota == 0, m, 0)
    pltpu.sync_copy(cnt_v, cnt_out.at[pl.ds(w * 16, 16)])


_prep = functools.partial(
    pl.kernel,
    out_type=[
        jax.ShapeDtypeStruct((NNP, D), jnp.float32),
        jax.ShapeDtypeStruct((NW * NNP,), jnp.float32),
        jax.ShapeDtypeStruct((NW * EPW,), jnp.int32),
        jax.ShapeDtypeStruct((NW * EPW,), jnp.int32),
        jax.ShapeDtypeStruct((NW * 2 * HCAP,), jnp.int32),
        jax.ShapeDtypeStruct((NW * 16,), jnp.int32),
        jax.ShapeDtypeStruct((T,), jnp.int32),
    ],
    mesh=_mesh,
    scratch_types=[
        pltpu.VMEM((NGC, GCH), jnp.int32),
        pltpu.VMEM((BPW, D), jnp.float32),
        pltpu.VMEM((EPW,), jnp.int32),
        pltpu.VMEM((EPW,), jnp.int32),
        pltpu.VMEM((NNP,), jnp.float32),
        pltpu.VMEM((NNP,), jnp.int32),
        pltpu.VMEM((EPW + 16,), jnp.int32),
        pltpu.VMEM((EPW + 16,), jnp.int32),
        pltpu.VMEM((16,), jnp.int32),
        pltpu.VMEM((T,), jnp.int32),
        pltpu.VMEM((T,), jnp.int32),
        pltpu.SemaphoreType.DMA,
        pltpu.SemaphoreType.DMA((3,)),
    ],
    compiler_params=_sc_params,
)(_prep_body)


def _frontier_body(l2s, head, cnts, n_idx, src_e, dst_e, zeros,
                   l1s_out, l1d_out, cnt_out,
                   fmask, head_v, cv, n_v, src_v, dst_v, cs, cd,
                   cnt_v, ovbuf):
    w = _wid()
    pltpu.sync_copy(zeros.at[pl.ds(0, NNP)], fmask)
    pltpu.sync_copy(n_idx, n_v)
    pltpu.sync_copy(cnts, cv)
    pltpu.sync_copy(head, head_v)
    ones = jnp.ones((16,), jnp.float32)
    for g in range(T // 16):
        t16 = n_v[pl.ds(g * 16, 16)]
        plsc.store_scatter(fmask, [t16], ones)
    for v in range(NW):
        m = jnp.sum(cv[pl.ds(v * 16, 16)])
        mm = jnp.minimum(m, HCAP)

        @plsc.parallel_loop(0, (mm + 15) // 16, unroll=2)
        def _sc(g, v=v):
            s16 = head_v[pl.ds(v * 2 * HCAP + g * 16, 16)]
            plsc.store_scatter(fmask, [s16], ones)

    def ov(v, c):
        m = jnp.sum(cv[pl.ds(v * 16, 16)])

        @pl.when(m > HCAP)
        def _():
            def part(p, c2):
                pltpu.sync_copy(l2s.at[pl.ds(v * EPW + p * PCH, PCH)], ovbuf)
                rem = jnp.minimum(m - p * PCH, PCH)

                def grp2(g, c3):
                    s16 = ovbuf[pl.ds(g * 16, 16)]
                    plsc.store_scatter(fmask, [s16], ones)
                    return c3

                lax.fori_loop(0, (rem + 15) // 16, grp2, 0)
                return c2

            lax.fori_loop(1, (m + PCH - 1) // PCH, part, 0)

        return c

    lax.fori_loop(0, NW, ov, 0)

    padv = jnp.full((16,), PAD, jnp.int32)

    @plsc.parallel_loop(0, (EPW + 16) // 16, unroll=8)
    def _fill(g):
        cs[pl.ds(g * 16, 16)] = padv
        cd[pl.ds(g * 16, 16)] = padv

    pltpu.sync_copy(src_e.at[pl.ds(w * EPW, EPW)], src_v)
    pltpu.sync_copy(dst_e.at[pl.ds(w * EPW, EPW)], dst_v)

    @plsc.parallel_loop(0, EPW // 16, unroll=4, carry=jnp.int32(0))
    def _cmp(g, m):
        off = g * 16
        s16 = src_v[pl.ds(off, 16)]
        d16 = dst_v[pl.ds(off, 16)]
        fv = plsc.load_gather(fmask, [d16])
        msk = fv > 0.0
        plsc.store_compressed(cs.at[pl.ds(m, 16)], s16, mask=msk)
        plsc.store_compressed(cd.at[pl.ds(m, 16)], d16, mask=msk)
        return m + jnp.sum(msk.astype(jnp.int32))

    m = _cmp
    pltpu.sync_copy(cs.at[pl.ds(0, EPW)], l1s_out.at[pl.ds(w * EPW, EPW)])
    pltpu.sync_copy(cd.at[pl.ds(0, EPW)], l1d_out.at[pl.ds(w * EPW, EPW)])
    iota = lax.iota(jnp.int32, 16)
    cnt_v[...] = jnp.where(iota == 0, m, 0)
    pltpu.sync_copy(cnt_v, cnt_out.at[pl.ds(w * 16, 16)])


_frontier = functools.partial(
    pl.kernel,
    out_type=[
        jax.ShapeDtypeStruct((NW * EPW,), jnp.int32),
        jax.ShapeDtypeStruct((NW * EPW,), jnp.int32),
        jax.ShapeDtypeStruct((NW * 16,), jnp.int32),
    ],
    mesh=_mesh,
    scratch_types=[
        pltpu.VMEM((NNP,), jnp.float32),
        pltpu.VMEM((NW * 2 * HCAP,), jnp.int32),
        pltpu.VMEM((NW * 16,), jnp.int32),
        pltpu.VMEM((T,), jnp.int32),
        pltpu.VMEM((EPW,), jnp.int32),
        pltpu.VMEM((EPW,), jnp.int32),
        pltpu.VMEM((EPW + 16,), jnp.int32),
        pltpu.VMEM((EPW + 16,), jnp.int32),
        pltpu.VMEM((16,), jnp.int32),
        pltpu.VMEM((PCH,), jnp.int32),
    ],
    compiler_params=_sc_params,
)(_frontier_body)


def _spmm_body(xw_t, l1s, l1d, cnts1, l2s, l2d, cnts2, n_idx, canon,
               dinv, b1, zeros, out,
               xs, acc, dinv_v, b1_v, src_v0, src_v1, dst_v0, dst_v1,
               cv, cv2, acc2, n_v2, canon_v, sel_v, sems, semd, psem):
    w = _wid()
    pdescs = [
        pltpu.async_copy(xw_t.at[pl.ds(w * FPW * NNP, FPW * NNP)], xs,
                         psem.at[0]),
        pltpu.async_copy(dinv, dinv_v, psem.at[1]),
        pltpu.async_copy(b1, b1_v, psem.at[2]),
        pltpu.async_copy(zeros, acc, psem.at[3]),
    ]
    pltpu.sync_copy(cnts1, cv)
    for d in pdescs:
        d.wait()

    bufs = ((src_v0, dst_v0), (src_v1, dst_v1))
    for b in range(2):
        pltpu.async_copy(l1s.at[pl.ds(b * EPW, CAPL)], bufs[b][0],
                         sems.at[b])
        pltpu.async_copy(l1d.at[pl.ds(b * EPW, CAPL)], bufs[b][1],
                         semd.at[b])

    @pl.loop(0, NW, step=2)
    def _chunks(k):
        for b in range(2):
            kk = k + b
            sv, dv_ = bufs[b]
            pltpu.make_async_copy(l1s.at[pl.ds(kk * EPW, CAPL)],
                                  sv, sems.at[b]).wait()
            pltpu.make_async_copy(l1d.at[pl.ds(kk * EPW, CAPL)],
                                  dv_, semd.at[b]).wait()
            m = jnp.sum(cv[pl.ds(kk * 16, 16)])
            mm = jnp.minimum(m, CAPL)

            @plsc.parallel_loop(0, (mm + 15) // 16, unroll=8)
            def _group(g):
                off = g * 16
                s16 = sv[pl.ds(off, 16)]
                d16 = dv_[pl.ds(off, 16)]
                nv = (plsc.load_gather(dinv_v, [s16])
                      * plsc.load_gather(dinv_v, [d16]))
                for j in range(FPW):
                    xv = plsc.load_gather(xs, [s16 + (j * NNP)])
                    plsc.addupdate_scatter(acc, [d16 + (j * NNP)], xv * nv)

            @pl.when(kk + 2 < NW)
            def _prefetch():
                pltpu.async_copy(l1s.at[pl.ds((kk + 2) * EPW, CAPL)],
                                 sv, sems.at[b])
                pltpu.async_copy(l1d.at[pl.ds((kk + 2) * EPW, CAPL)],
                                 dv_, semd.at[b])

    def ov(v, c):
        m = jnp.sum(cv[pl.ds(v * 16, 16)])

        @pl.when(m > CAPL)
        def _():
            def part(p, c2):
                off0 = v * EPW + CAPL + p * PCH
                pltpu.sync_copy(l1s.at[pl.ds(off0, PCH)],
                                src_v0.at[pl.ds(0, PCH)])
                pltpu.sync_copy(l1d.at[pl.ds(off0, PCH)],
                                dst_v0.at[pl.ds(0, PCH)])
                rem = jnp.minimum(m - CAPL - p * PCH, PCH)

                def grp2(g, c3):
                    off = g * 16
                    s16 = src_v0[pl.ds(off, 16)]
                    d16 = dst_v0[pl.ds(off, 16)]
                    nv = (plsc.load_gather(dinv_v, [s16])
                          * plsc.load_gather(dinv_v, [d16]))
                    for j in range(FPW):
                        xv = plsc.load_gather(xs, [s16 + (j * NNP)])
                        plsc.addupdate_scatter(acc, [d16 + (j * NNP)],
                                               xv * nv)
                    return c3

                lax.fori_loop(0, (rem + 15) // 16, grp2, 0)
                return c2

            lax.fori_loop(0, (m - CAPL + PCH - 1) // PCH, part, 0)

        return c

    lax.fori_loop(0, NW, ov, 0)

    @plsc.parallel_loop(0, NNP // 16, unroll=4)
    def _ep(g):
        off = g * 16
        dv = dinv_v[pl.ds(off, 16)]
        d2 = dv * dv
        for j in range(FPW):
            o = j * NNP + off
            bj = plsc.load_gather(b1_v, [jnp.full((16,), w * FPW + j,
                                                  jnp.int32)])
            v = acc[pl.ds(o, 16)] + d2 * xs[pl.ds(o, 16)] + bj
            acc[pl.ds(o, 16)] = jnp.maximum(v, 0.15 * v)

    pltpu.sync_copy(cnts2, cv2)
    pltpu.sync_copy(n_idx, n_v2.at[pl.ds(0, T)])
    pltpu.sync_copy(canon, canon_v)
    padv = jnp.full((16,), PAD, jnp.int32)
    n_v2[pl.ds(T, 16)] = padv

    @plsc.parallel_loop(0, (FPW * TST + 15) // 16, unroll=2)
    def _z2(g):
        acc2[pl.ds(g * 16, 16)] = jnp.zeros((16,), jnp.float32)

    def l2_group(sref, sbase, pref, pbase, g):
        s16 = sref[pl.ds(sbase + g * 16, 16)]
        p16 = pref[pl.ds(pbase + g * 16, 16)]
        dn16 = plsc.load_gather(n_v2, [p16])
        nv = (plsc.load_gather(dinv_v, [s16])
              * plsc.load_gather(dinv_v, [dn16]))
        for j in range(FPW):
            hv = plsc.load_gather(acc, [s16 + (j * NNP)])
            plsc.addupdate_scatter(acc2, [p16 + (j * TST)], hv * nv)

    for b in range(2):
        pltpu.async_copy(l2s.at[pl.ds(b * EPW, HCAP)],
                         bufs[b][0].at[pl.ds(0, HCAP)], sems.at[b])
        pltpu.async_copy(l2d.at[pl.ds(b * EPW, HCAP)],
                         bufs[b][1].at[pl.ds(0, HCAP)], semd.at[b])

    @pl.loop(0, NW, step=2)
    def _l2chunks(k):
        for b in range(2):
            kk = k + b
            sv, dv_ = bufs[b]
            pltpu.make_async_copy(l2s.at[pl.ds(kk * EPW, HCAP)],
                                  sv.at[pl.ds(0, HCAP)], sems.at[b]).wait()
            pltpu.make_async_copy(l2d.at[pl.ds(kk * EPW, HCAP)],
                                  dv_.at[pl.ds(0, HCAP)], semd.at[b]).wait()
            m = jnp.sum(cv2[pl.ds(kk * 16, 16)])
            mm = jnp.minimum(m, HCAP)

            @plsc.parallel_loop(0, (mm + 15) // 16, unroll=4)
            def _g2(g):
                l2_group(sv, 0, dv_, 0, g)

            @pl.when(kk + 2 < NW)
            def _pf2():
                pltpu.async_copy(l2s.at[pl.ds((kk + 2) * EPW, HCAP)],
                                 sv.at[pl.ds(0, HCAP)], sems.at[b])
                pltpu.async_copy(l2d.at[pl.ds((kk + 2) * EPW, HCAP)],
                                 dv_.at[pl.ds(0, HCAP)], semd.at[b])

    def ov2(v, c):
        m = jnp.sum(cv2[pl.ds(v * 16, 16)])

        @pl.when(m > HCAP)
        def _():
            def part(p, c2):
                off0 = v * EPW + HCAP + p * PCH
                pltpu.sync_copy(l2s.at[pl.ds(off0, PCH)],
                                src_v0.at[pl.ds(0, PCH)])
                pltpu.sync_copy(l2d.at[pl.ds(off0, PCH)],
                                dst_v0.at[pl.ds(0, PCH)])
                rem = jnp.minimum(m - HCAP - p * PCH, PCH)

                def grp2(g, c3):
                    l2_group(src_v0, 0, dst_v0, 0, g)
                    return c3

                lax.fori_loop(0, (rem + 15) // 16, grp2, 0)
                return c2

            lax.fori_loop(0, (m - HCAP + PCH - 1) // PCH, part, 0)

        return c

    lax.fori_loop(0, NW, ov2, 0)

    for g in range(T // 16):
        t16 = n_v2[pl.ds(g * 16, 16)]
        c16 = canon_v[pl.ds(g * 16, 16)]
        dv = plsc.load_gather(dinv_v, [t16])
        d2 = dv * dv
        for j in range(FPW):
            av = plsc.load_gather(acc2, [c16 + (j * TST)])
            hv = plsc.load_gather(acc, [t16 + (j * NNP)])
            sel_v[pl.ds(j * T + g * 16, 16)] = av + d2 * hv
    pltpu.sync_copy(sel_v, out.at[pl.ds(w * FPW * T, FPW * T)])


_spmm_full = functools.partial(
    pl.kernel,
    out_type=jax.ShapeDtypeStruct((NW * FPW * T,), jnp.float32),
    mesh=_mesh,
    scratch_types=[
        pltpu.VMEM((FPW * NNP,), jnp.float32),
        pltpu.VMEM((FPW * NNP,), jnp.float32),
        pltpu.VMEM((NNP,), jnp.float32),
        pltpu.VMEM((D,), jnp.float32),
        pltpu.VMEM((CAPL,), jnp.int32),
        pltpu.VMEM((CAPL,), jnp.int32),
        pltpu.VMEM((CAPL,), jnp.int32),
        pltpu.VMEM((CAPL,), jnp.int32),
        pltpu.VMEM((NW * 16,), jnp.int32),
        pltpu.VMEM((NW * 16,), jnp.int32),
        pltpu.VMEM((FPW * TST,), jnp.float32),
        pltpu.VMEM((T + 16,), jnp.int32),
        pltpu.VMEM((T,), jnp.int32),
        pltpu.VMEM((FPW * T,), jnp.float32),
        pltpu.SemaphoreType.DMA((2,)),
        pltpu.SemaphoreType.DMA((2,)),
        pltpu.SemaphoreType.DMA((4,)),
    ],
    compiler_params=_sc_params,
)(_spmm_body)


def _ln_w1_body(rows_ref, w1_ref, degp_ref, out_ref, dinv_ref):
    r = rows_ref[:]
    mu = jnp.mean(r, axis=-1, keepdims=True)
    var = jnp.mean((r - mu) ** 2, axis=-1, keepdims=True)
    x = (r - mu) * lax.rsqrt(var + 1e-5)
    out_ref[:] = lax.dot_general(
        w1_ref[:], x, (((1,), (1,)), ((), ())),
        preferred_element_type=jnp.float32)

    @pl.when(pl.program_id(0) == 0)
    def _():
        deg = jnp.sum(degp_ref[:], axis=0) + 1.0
        dinv_ref[:] = lax.rsqrt(deg)


def _head_body(sel_ref, w2_ref, b2_ref, wout_ref, bout_ref, out_ref,
               tmp_ref):
    @pl.when(pl.program_id(0) == 0)
    def _():
        tmp_ref[:] = lax.dot_general(
            sel_ref[:], w2_ref[:], (((0,), (1,)), ((), ())),
            preferred_element_type=jnp.float32) + b2_ref[:]

    out_ref[:] = lax.dot_general(
        tmp_ref[:], wout_ref[:], (((1,), (1,)), ((), ())),
        preferred_element_type=jnp.float32) + bout_ref[:]


_NB = 1024
_VB = 4096


def kernel(edge_index, N, y, emb, W1, b1, W2, b2, Wout, bout):
    src = edge_index[0].astype(jnp.int32)
    dst = edge_index[1].astype(jnp.int32)
    y_pad = jnp.concatenate(
        [y.astype(jnp.int32), jnp.zeros((NNP - NN,), jnp.int32)]
    ).reshape(NW, NGC, GCH)
    n_idx = N.astype(jnp.int32)
    zeros = jnp.zeros((FPW * NNP,), jnp.float32)

    rows, deg_p, l2s, l2d, head, cnts, canon = _prep(emb, y_pad, src, dst,
                                                     n_idx, zeros)
    l1s, l1d, cnts1 = _frontier(l2s, head, cnts, n_idx, src, dst, zeros)

    xw1_t, dinv = pl.pallas_call(
        _ln_w1_body,
        grid=(NNP // _NB,),
        in_specs=[
            pl.BlockSpec((_NB, D), lambda i: (i, 0)),
            pl.BlockSpec((D, D), lambda i: (0, 0)),
            pl.BlockSpec((NW, NNP // D, D), lambda i: (0, 0, 0)),
        ],
        out_specs=[
            pl.BlockSpec((D, _NB), lambda i: (0, i)),
            pl.BlockSpec((NNP // D, D), lambda i: (0, 0)),
        ],
        out_shape=[
            jax.ShapeDtypeStruct((D, NNP), jnp.float32),
            jax.ShapeDtypeStruct((NNP // D, D), jnp.float32),
        ],
    )(rows, W1, deg_p.reshape(NW, NNP // D, D))
    dinv = dinv.reshape(NNP)

    sel = _spmm_full(xw1_t.reshape(D * NNP), l1s, l1d, cnts1,
                     l2s, l2d, cnts, n_idx, canon, dinv, b1, zeros)

    out = pl.pallas_call(
        _head_body,
        grid=(pl.cdiv(V, _VB),),
        in_specs=[
            pl.BlockSpec((D, T), lambda i: (0, 0)),
            pl.BlockSpec((D, D), lambda i: (0, 0)),
            pl.BlockSpec((1, D), lambda i: (0, 0)),
            pl.BlockSpec((_VB, D), lambda i: (i, 0)),
            pl.BlockSpec((1, _VB), lambda i: (0, i)),
        ],
        out_specs=pl.BlockSpec((T, _VB), lambda i: (0, i)),
        out_shape=jax.ShapeDtypeStruct((T, V), jnp.float32),
        scratch_shapes=[pltpu.VMEM((T, D), jnp.float32)],
    )(sel.reshape(NW * FPW, T).reshape(D, T), W2, b2.reshape(1, D),
      Wout, bout.reshape(1, V))

    return out


# speedup vs baseline: 1.1065x; 1.0186x over previous
import functools

import jax
import jax.numpy as jnp
from jax import lax
from jax.experimental import pallas as pl
from jax.experimental.pallas import tpu as pltpu
from jax.experimental.pallas import tpu_sc as plsc

NN = 10000
NNP = 10240
E = 320000
D = 128
V = 100000
T = 128
NC, NS = 2, 16
NW = NC * NS
FPW = D // NW
BPW = NNP // NW
GCH = 64
NGC = BPW // GCH
EPW = E // NW
ECH = 8000
NEC = E // ECH
HCAP = 512
PCH = 512
CAPL = 8192
PAD = NNP - 1
TPAD = T
TST = T + 8

_mesh = plsc.VectorSubcoreMesh(
    core_axis_name="c", subcore_axis_name="s", num_cores=NC, num_subcores=NS
)
_sc_params = pltpu.CompilerParams(needs_layout_passes=False)


def _wid():
    return lax.axis_index("s") * NC + lax.axis_index("c")


def _prep_body(emb, y_r, src_e, dst_e, n_idx, zeros,
               rows_out, degp_out, l2s_out, l2d_out, head_out, cnt_out,
               canon_out,
               idx_v, rows_v, src_v, dst_v, deg_v, tmask, cs, cd,
               cnt_v, n_v, canon_v, sem, esem):
    w = _wid()
    base = w * BPW
    pltpu.sync_copy(y_r.at[w], idx_v)
    edescs = [
        pltpu.async_copy(src_e.at[pl.ds(w * EPW, EPW)], src_v, esem.at[0]),
        pltpu.async_copy(dst_e.at[pl.ds(w * EPW, EPW)], dst_v, esem.at[1]),
        pltpu.async_copy(zeros.at[pl.ds(0, NNP)], deg_v, esem.at[2]),
    ]
    descs = [pltpu.async_copy(emb.at[idx_v.at[q]],
                              rows_v.at[pl.ds(q * GCH, GCH)], sem)
             for q in range(NGC)]
    for d in descs:
        d.wait()
    pltpu.sync_copy(rows_v, rows_out.at[pl.ds(base, BPW)])

    neg1 = jnp.full((16,), -1, jnp.int32)

    @plsc.parallel_loop(0, NNP // 16, unroll=8)
    def _tinit(g):
        tmask[pl.ds(g * 16, 16)] = neg1

    pltpu.sync_copy(n_idx, n_v)
    ones = jnp.ones((16,), jnp.float32)
    iota = lax.iota(jnp.int32, 16)
    for g in range(T // 16):
        t16 = n_v[pl.ds(g * 16, 16)]
        plsc.store_scatter(tmask, [t16], iota + (g * 16))
    for g in range(T // 16):
        t16 = n_v[pl.ds(g * 16, 16)]
        canon_v[pl.ds(g * 16, 16)] = plsc.load_gather(tmask, [t16])

    padv = jnp.full((16,), PAD, jnp.int32)
    tpadv = jnp.full((16,), TPAD, jnp.int32)

    @plsc.parallel_loop(0, (EPW + 16) // 16, unroll=8)
    def _fill(g):
        cs[pl.ds(g * 16, 16)] = padv
        cd[pl.ds(g * 16, 16)] = tpadv

    for d in edescs:
        d.wait()

    @plsc.parallel_loop(0, EPW // 16, unroll=4, carry=jnp.int32(0))
    def _deg(g, m):
        off = g * 16
        s16 = src_v[pl.ds(off, 16)]
        d16 = dst_v[pl.ds(off, 16)]
        plsc.addupdate_scatter(deg_v, [d16], ones)
        pv = plsc.load_gather(tmask, [d16])
        msk = pv >= 0
        plsc.store_compressed(cs.at[pl.ds(m, 16)], s16, mask=msk)
        plsc.store_compressed(cd.at[pl.ds(m, 16)], pv, mask=msk)
        return m + jnp.sum(msk.astype(jnp.int32))

    m = _deg
    pltpu.sync_copy(deg_v, degp_out.at[pl.ds(w * NNP, NNP)])
    pltpu.sync_copy(cs.at[pl.ds(0, EPW)], l2s_out.at[pl.ds(w * EPW, EPW)])
    pltpu.sync_copy(cd.at[pl.ds(0, EPW)], l2d_out.at[pl.ds(w * EPW, EPW)])
    hb = w * 2 * HCAP
    pltpu.sync_copy(cs.at[pl.ds(0, HCAP)], head_out.at[pl.ds(hb, HCAP)])
    pltpu.sync_copy(cd.at[pl.ds(0, HCAP)], head_out.at[pl.ds(hb + HCAP, HCAP)])
    @pl.when(w == 0)
    def _():
        pltpu.sync_copy(canon_v, canon_out)

    cnt_v[...] = jnp.where(iota == 0, m, 0)
    pltpu.sync_copy(cnt_v, cnt_out.at[pl.ds(w * 16, 16)])


_prep = functools.partial(
    pl.kernel,
    out_type=[
        jax.ShapeDtypeStruct((NNP, D), jnp.float32),
        jax.ShapeDtypeStruct((NW * NNP,), jnp.float32),
        jax.ShapeDtypeStruct((NW * EPW,), jnp.int32),
        jax.ShapeDtypeStruct((NW * EPW,), jnp.int32),
        jax.ShapeDtypeStruct((NW * 2 * HCAP,), jnp.int32),
        jax.ShapeDtypeStruct((NW * 16,), jnp.int32),
        jax.ShapeDtypeStruct((T,), jnp.int32),
    ],
    mesh=_mesh,
    scratch_types=[
        pltpu.VMEM((NGC, GCH), jnp.int32),
        pltpu.VMEM((BPW, D), jnp.float32),
        pltpu.VMEM((EPW,), jnp.int32),
        pltpu.VMEM((EPW,), jnp.int32),
        pltpu.VMEM((NNP,), jnp.float32),
        pltpu.VMEM((NNP,), jnp.int32),
        pltpu.VMEM((EPW + 16,), jnp.int32),
        pltpu.VMEM((EPW + 16,), jnp.int32),
        pltpu.VMEM((16,), jnp.int32),
        pltpu.VMEM((T,), jnp.int32),
        pltpu.VMEM((T,), jnp.int32),
        pltpu.SemaphoreType.DMA,
        pltpu.SemaphoreType.DMA((3,)),
    ],
    compiler_params=_sc_params,
)(_prep_body)


def _frontier_body(l2s, head, cnts, n_idx, src_e, dst_e, zeros,
                   l1s_out, l1d_out, cnt_out,
                   fmask, head_v, cv, n_v, src_v, dst_v, cs, cd,
                   cnt_v, ovbuf):
    w = _wid()
    pltpu.sync_copy(zeros.at[pl.ds(0, NNP)], fmask)
    pltpu.sync_copy(n_idx, n_v)
    pltpu.sync_copy(cnts, cv)
    pltpu.sync_copy(head, head_v)
    ones = jnp.ones((16,), jnp.float32)
    for g in range(T // 16):
        t16 = n_v[pl.ds(g * 16, 16)]
        plsc.store_scatter(fmask, [t16], ones)
    for v in range(NW):
        m = jnp.sum(cv[pl.ds(v * 16, 16)])
        mm = jnp.minimum(m, HCAP)

        @plsc.parallel_loop(0, (mm + 15) // 16, unroll=2)
        def _sc(g, v=v):
            s16 = head_v[pl.ds(v * 2 * HCAP + g * 16, 16)]
            plsc.store_scatter(fmask, [s16], ones)

    def ov(v, c):
        m = jnp.sum(cv[pl.ds(v * 16, 16)])

        @pl.when(m > HCAP)
        def _():
            def part(p, c2):
                pltpu.sync_copy(l2s.at[pl.ds(v * EPW + p * PCH, PCH)], ovbuf)
                rem = jnp.minimum(m - p * PCH, PCH)

                def grp2(g, c3):
                    s16 = ovbuf[pl.ds(g * 16, 16)]
                    plsc.store_scatter(fmask, [s16], ones)
                    return c3

                lax.fori_loop(0, (rem + 15) // 16, grp2, 0)
                return c2

            lax.fori_loop(1, (m + PCH - 1) // PCH, part, 0)

        return c

    lax.fori_loop(0, NW, ov, 0)

    padv = jnp.full((16,), PAD, jnp.int32)

    @plsc.parallel_loop(0, (EPW + 16) // 16, unroll=8)
    def _fill(g):
        cs[pl.ds(g * 16, 16)] = padv
        cd[pl.ds(g * 16, 16)] = padv

    pltpu.sync_copy(src_e.at[pl.ds(w * EPW, EPW)], src_v)
    pltpu.sync_copy(dst_e.at[pl.ds(w * EPW, EPW)], dst_v)

    @plsc.parallel_loop(0, EPW // 16, unroll=4, carry=jnp.int32(0))
    def _cmp(g, m):
        off = g * 16
        s16 = src_v[pl.ds(off, 16)]
        d16 = dst_v[pl.ds(off, 16)]
        fv = plsc.load_gather(fmask, [d16])
        msk = fv > 0.0
        plsc.store_compressed(cs.at[pl.ds(m, 16)], s16, mask=msk)
        plsc.store_compressed(cd.at[pl.ds(m, 16)], d16, mask=msk)
        return m + jnp.sum(msk.astype(jnp.int32))

    m = _cmp
    pltpu.sync_copy(cs.at[pl.ds(0, EPW)], l1s_out.at[pl.ds(w * EPW, EPW)])
    pltpu.sync_copy(cd.at[pl.ds(0, EPW)], l1d_out.at[pl.ds(w * EPW, EPW)])
    iota = lax.iota(jnp.int32, 16)
    cnt_v[...] = jnp.where(iota == 0, m, 0)
    pltpu.sync_copy(cnt_v, cnt_out.at[pl.ds(w * 16, 16)])


_frontier = functools.partial(
    pl.kernel,
    out_type=[
        jax.ShapeDtypeStruct((NW * EPW,), jnp.int32),
        jax.ShapeDtypeStruct((NW * EPW,), jnp.int32),
        jax.ShapeDtypeStruct((NW * 16,), jnp.int32),
    ],
    mesh=_mesh,
    scratch_types=[
        pltpu.VMEM((NNP,), jnp.float32),
        pltpu.VMEM((NW * 2 * HCAP,), jnp.int32),
        pltpu.VMEM((NW * 16,), jnp.int32),
        pltpu.VMEM((T,), jnp.int32),
        pltpu.VMEM((EPW,), jnp.int32),
        pltpu.VMEM((EPW,), jnp.int32),
        pltpu.VMEM((EPW + 16,), jnp.int32),
        pltpu.VMEM((EPW + 16,), jnp.int32),
        pltpu.VMEM((16,), jnp.int32),
        pltpu.VMEM((PCH,), jnp.int32),
    ],
    compiler_params=_sc_params,
)(_frontier_body)


def _spmm_body(xw_t, l1s, l1d, cnts1, l2s, l2d, cnts2, n_idx, canon,
               dinv, b1, zeros, out,
               xs, acc, dinv_v, b1_v, src_v0, src_v1, dst_v0, dst_v1,
               cv, cv2, acc2, n_v2, canon_v, sel_v, sems, semd, psem):
    w = _wid()
    pdescs = [
        pltpu.async_copy(xw_t.at[pl.ds(w * FPW * NNP, FPW * NNP)], xs,
                         psem.at[0]),
        pltpu.async_copy(dinv, dinv_v, psem.at[1]),
        pltpu.async_copy(b1, b1_v, psem.at[2]),
        pltpu.async_copy(zeros, acc, psem.at[3]),
    ]
    pltpu.sync_copy(cnts1, cv)
    for d in pdescs:
        d.wait()

    bufs = ((src_v0, dst_v0), (src_v1, dst_v1))
    for b in range(2):
        pltpu.async_copy(l1s.at[pl.ds(b * EPW, CAPL)], bufs[b][0],
                         sems.at[b])
        pltpu.async_copy(l1d.at[pl.ds(b * EPW, CAPL)], bufs[b][1],
                         semd.at[b])

    @pl.loop(0, NW, step=2)
    def _chunks(k):
        for b in range(2):
            kk = k + b
            sv, dv_ = bufs[b]
            pltpu.make_async_copy(l1s.at[pl.ds(kk * EPW, CAPL)],
                                  sv, sems.at[b]).wait()
            pltpu.make_async_copy(l1d.at[pl.ds(kk * EPW, CAPL)],
                                  dv_, semd.at[b]).wait()
            m = jnp.sum(cv[pl.ds(kk * 16, 16)])
            mm = jnp.minimum(m, CAPL)

            @plsc.parallel_loop(0, (mm + 15) // 16, unroll=8)
            def _group(g):
                off = g * 16
                s16 = sv[pl.ds(off, 16)]
                d16 = dv_[pl.ds(off, 16)]
                nv = (plsc.load_gather(dinv_v, [s16])
                      * plsc.load_gather(dinv_v, [d16]))
                for j in range(FPW):
                    xv = plsc.load_gather(xs, [s16 + (j * NNP)])
                    plsc.addupdate_scatter(acc, [d16 + (j * NNP)], xv * nv)

            @pl.when(kk + 2 < NW)
            def _prefetch():
                pltpu.async_copy(l1s.at[pl.ds((kk + 2) * EPW, CAPL)],
                                 sv, sems.at[b])
                pltpu.async_copy(l1d.at[pl.ds((kk + 2) * EPW, CAPL)],
                                 dv_, semd.at[b])

    def ov(v, c):
        m = jnp.sum(cv[pl.ds(v * 16, 16)])

        @pl.when(m > CAPL)
        def _():
            def part(p, c2):
                off0 = v * EPW + CAPL + p * PCH
                pltpu.sync_copy(l1s.at[pl.ds(off0, PCH)],
                                src_v0.at[pl.ds(0, PCH)])
                pltpu.sync_copy(l1d.at[pl.ds(off0, PCH)],
                                dst_v0.at[pl.ds(0, PCH)])
                rem = jnp.minimum(m - CAPL - p * PCH, PCH)

                def grp2(g, c3):
                    off = g * 16
                    s16 = src_v0[pl.ds(off, 16)]
                    d16 = dst_v0[pl.ds(off, 16)]
                    nv = (plsc.load_gather(dinv_v, [s16])
                          * plsc.load_gather(dinv_v, [d16]))
                    for j in range(FPW):
                        xv = plsc.load_gather(xs, [s16 + (j * NNP)])
                        plsc.addupdate_scatter(acc, [d16 + (j * NNP)],
                                               xv * nv)
                    return c3

                lax.fori_loop(0, (rem + 15) // 16, grp2, 0)
                return c2

            lax.fori_loop(0, (m - CAPL + PCH - 1) // PCH, part, 0)

        return c

    lax.fori_loop(0, NW, ov, 0)

    @plsc.parallel_loop(0, NNP // 16, unroll=4)
    def _ep(g):
        off = g * 16
        dv = dinv_v[pl.ds(off, 16)]
        d2 = dv * dv
        for j in range(FPW):
            o = j * NNP + off
            bj = plsc.load_gather(b1_v, [jnp.full((16,), w * FPW + j,
                                                  jnp.int32)])
            v = acc[pl.ds(o, 16)] + d2 * xs[pl.ds(o, 16)] + bj
            acc[pl.ds(o, 16)] = jnp.maximum(v, 0.15 * v)

    pltpu.sync_copy(cnts2, cv2)
    pltpu.sync_copy(n_idx, n_v2.at[pl.ds(0, T)])
    pltpu.sync_copy(canon, canon_v)
    padv = jnp.full((16,), PAD, jnp.int32)
    n_v2[pl.ds(T, 16)] = padv

    @plsc.parallel_loop(0, (FPW * TST + 15) // 16, unroll=2)
    def _z2(g):
        acc2[pl.ds(g * 16, 16)] = jnp.zeros((16,), jnp.float32)

    def l2_group(sref, sbase, pref, pbase, g):
        s16 = sref[pl.ds(sbase + g * 16, 16)]
        p16 = pref[pl.ds(pbase + g * 16, 16)]
        dn16 = plsc.load_gather(n_v2, [p16])
        nv = (plsc.load_gather(dinv_v, [s16])
              * plsc.load_gather(dinv_v, [dn16]))
        for j in range(FPW):
            hv = plsc.load_gather(acc, [s16 + (j * NNP)])
            plsc.addupdate_scatter(acc2, [p16 + (j * TST)], hv * nv)

    for b in range(2):
        pltpu.async_copy(l2s.at[pl.ds(b * EPW, HCAP)],
                         bufs[b][0].at[pl.ds(0, HCAP)], sems.at[b])
        pltpu.async_copy(l2d.at[pl.ds(b * EPW, HCAP)],
                         bufs[b][1].at[pl.ds(0, HCAP)], semd.at[b])

    @pl.loop(0, NW, step=2)
    def _l2chunks(k):
        for b in range(2):
            kk = k + b
            sv, dv_ = bufs[b]
            pltpu.make_async_copy(l2s.at[pl.ds(kk * EPW, HCAP)],
                                  sv.at[pl.ds(0, HCAP)], sems.at[b]).wait()
            pltpu.make_async_copy(l2d.at[pl.ds(kk * EPW, HCAP)],
                                  dv_.at[pl.ds(0, HCAP)], semd.at[b]).wait()
            m = jnp.sum(cv2[pl.ds(kk * 16, 16)])
            mm = jnp.minimum(m, HCAP)

            @plsc.parallel_loop(0, (mm + 15) // 16, unroll=4)
            def _g2(g):
                l2_group(sv, 0, dv_, 0, g)

            @pl.when(kk + 2 < NW)
            def _pf2():
                pltpu.async_copy(l2s.at[pl.ds((kk + 2) * EPW, HCAP)],
                                 sv.at[pl.ds(0, HCAP)], sems.at[b])
                pltpu.async_copy(l2d.at[pl.ds((kk + 2) * EPW, HCAP)],
                                 dv_.at[pl.ds(0, HCAP)], semd.at[b])

    def ov2(v, c):
        m = jnp.sum(cv2[pl.ds(v * 16, 16)])

        @pl.when(m > HCAP)
        def _():
            def part(p, c2):
                off0 = v * EPW + HCAP + p * PCH
                pltpu.sync_copy(l2s.at[pl.ds(off0, PCH)],
                                src_v0.at[pl.ds(0, PCH)])
                pltpu.sync_copy(l2d.at[pl.ds(off0, PCH)],
                                dst_v0.at[pl.ds(0, PCH)])
                rem = jnp.minimum(m - HCAP - p * PCH, PCH)

                def grp2(g, c3):
                    l2_group(src_v0, 0, dst_v0, 0, g)
                    return c3

                lax.fori_loop(0, (rem + 15) // 16, grp2, 0)
                return c2

            lax.fori_loop(0, (m - HCAP + PCH - 1) // PCH, part, 0)

        return c

    lax.fori_loop(0, NW, ov2, 0)

    for g in range(T // 16):
        t16 = n_v2[pl.ds(g * 16, 16)]
        c16 = canon_v[pl.ds(g * 16, 16)]
        dv = plsc.load_gather(dinv_v, [t16])
        d2 = dv * dv
        for j in range(FPW):
            av = plsc.load_gather(acc2, [c16 + (j * TST)])
            hv = plsc.load_gather(acc, [t16 + (j * NNP)])
            sel_v[pl.ds(j * T + g * 16, 16)] = av + d2 * hv
    pltpu.sync_copy(sel_v, out.at[pl.ds(w * FPW * T, FPW * T)])


_spmm_full = functools.partial(
    pl.kernel,
    out_type=jax.ShapeDtypeStruct((NW * FPW * T,), jnp.float32),
    mesh=_mesh,
    scratch_types=[
        pltpu.VMEM((FPW * NNP,), jnp.float32),
        pltpu.VMEM((FPW * NNP,), jnp.float32),
        pltpu.VMEM((NNP,), jnp.float32),
        pltpu.VMEM((D,), jnp.float32),
        pltpu.VMEM((CAPL,), jnp.int32),
        pltpu.VMEM((CAPL,), jnp.int32),
        pltpu.VMEM((CAPL,), jnp.int32),
        pltpu.VMEM((CAPL,), jnp.int32),
        pltpu.VMEM((NW * 16,), jnp.int32),
        pltpu.VMEM((NW * 16,), jnp.int32),
        pltpu.VMEM((FPW * TST,), jnp.float32),
        pltpu.VMEM((T + 16,), jnp.int32),
        pltpu.VMEM((T,), jnp.int32),
        pltpu.VMEM((FPW * T,), jnp.float32),
        pltpu.SemaphoreType.DMA((2,)),
        pltpu.SemaphoreType.DMA((2,)),
        pltpu.SemaphoreType.DMA((4,)),
    ],
    compiler_params=_sc_params,
)(_spmm_body)


def _ln_w1_body(rows_ref, w1_ref, degp_ref, out_ref, dinv_ref):
    r = rows_ref[:]
    mu = jnp.mean(r, axis=-1, keepdims=True)
    var = jnp.mean((r - mu) ** 2, axis=-1, keepdims=True)
    x = (r - mu) * lax.rsqrt(var + 1e-5)
    out_ref[:] = lax.dot_general(
        w1_ref[:], x, (((1,), (1,)), ((), ())),
        preferred_element_type=jnp.float32)

    @pl.when(pl.program_id(0) == 0)
    def _():
        deg = jnp.sum(degp_ref[:], axis=0) + 1.0
        dinv_ref[:] = lax.rsqrt(deg)


def _head_body(sel_ref, w2_ref, b2_ref, wout_ref, bout_ref, out_ref,
               tmp_ref):
    @pl.when(pl.program_id(0) == 0)
    def _():
        tmp_ref[:] = lax.dot_general(
            sel_ref[:], w2_ref[:], (((0,), (1,)), ((), ())),
            preferred_element_type=jnp.float32) + b2_ref[:]

    out_ref[:] = lax.dot_general(
        tmp_ref[:], wout_ref[:], (((1,), (1,)), ((), ())),
        preferred_element_type=jnp.float32) + bout_ref[:]


_NB = 1024
_VB = 8192


def kernel(edge_index, N, y, emb, W1, b1, W2, b2, Wout, bout):
    src = edge_index[0].astype(jnp.int32)
    dst = edge_index[1].astype(jnp.int32)
    y_pad = jnp.concatenate(
        [y.astype(jnp.int32), jnp.zeros((NNP - NN,), jnp.int32)]
    ).reshape(NW, NGC, GCH)
    n_idx = N.astype(jnp.int32)
    zeros = jnp.zeros((FPW * NNP,), jnp.float32)

    rows, deg_p, l2s, l2d, head, cnts, canon = _prep(emb, y_pad, src, dst,
                                                     n_idx, zeros)
    l1s, l1d, cnts1 = _frontier(l2s, head, cnts, n_idx, src, dst, zeros)

    xw1_t, dinv = pl.pallas_call(
        _ln_w1_body,
        grid=(NNP // _NB,),
        in_specs=[
            pl.BlockSpec((_NB, D), lambda i: (i, 0)),
            pl.BlockSpec((D, D), lambda i: (0, 0)),
            pl.BlockSpec((NW, NNP // D, D), lambda i: (0, 0, 0)),
        ],
        out_specs=[
            pl.BlockSpec((D, _NB), lambda i: (0, i)),
            pl.BlockSpec((NNP // D, D), lambda i: (0, 0)),
        ],
        out_shape=[
            jax.ShapeDtypeStruct((D, NNP), jnp.float32),
            jax.ShapeDtypeStruct((NNP // D, D), jnp.float32),
        ],
    )(rows, W1, deg_p.reshape(NW, NNP // D, D))
    dinv = dinv.reshape(NNP)

    sel = _spmm_full(xw1_t.reshape(D * NNP), l1s, l1d, cnts1,
                     l2s, l2d, cnts, n_idx, canon, dinv, b1, zeros)

    out = pl.pallas_call(
        _head_body,
        grid=(pl.cdiv(V, _VB),),
        in_specs=[
            pl.BlockSpec((D, T), lambda i: (0, 0)),
            pl.BlockSpec((D, D), lambda i: (0, 0)),
            pl.BlockSpec((1, D), lambda i: (0, 0)),
            pl.BlockSpec((_VB, D), lambda i: (i, 0)),
            pl.BlockSpec((1, _VB), lambda i: (0, i)),
        ],
        out_specs=pl.BlockSpec((T, _VB), lambda i: (0, i)),
        out_shape=jax.ShapeDtypeStruct((T, V), jnp.float32),
        scratch_shapes=[pltpu.VMEM((T, D), jnp.float32)],
    )(sel.reshape(NW * FPW, T).reshape(D, T), W2, b2.reshape(1, D),
      Wout, bout.reshape(1, V))

    return out
